# baseline (speedup 1.0000x reference)
import functools

import jax
import jax.numpy as jnp
import numpy as np
from jax import lax
from jax.experimental import pallas as pl
from jax.experimental.pallas import tpu as pltpu
from jax.experimental.pallas import tpu_sc as plsc

_VERT_IDX = np.array([
    4646, 4779, 5143, 5109, 5112, 3207, 2391, 5398, 5786, 5925, 5831,
    5895, 2158, 6208, 6428, 6585, 6615, 6620, 2039, 2828, 6783, 7158,
    7407, 7308, 7368, 3820, 3536, 7707, 7856, 8051, 8056, 8063, 5669,
    5891, 5780, 5740, 6468, 6554, 6412, 6297, 7214, 7389, 7122, 7144,
    7975, 8059
], dtype=np.int64)

_B = 4096
_V = 8064
_C = 3
_A = 46
_L = 16
_ROWS_PER_BATCH = _V * _C // _L
_NW = 32
_B_PER_W = _B // _NW
_CB = 8
_NCHUNK = _B_PER_W // _CB
_OUT_PER_B = _A * _C
_OUT_PER_CHUNK = _CB * _OUT_PER_B
_NVEC = _OUT_PER_CHUNK // _L
_OUT_PER_W = _B_PER_W * _OUT_PER_B


def _build_static():
    elems = (_VERT_IDX[:, None] * _C + np.arange(_C)[None, :]).ravel()
    rows_u = np.unique(elems // _L)
    K = len(rows_u)
    pos_of = {int(r): p for p, r in enumerate(rows_u)}

    nidx = _CB * K
    nslots = 512
    gidx = np.zeros((_NW, _NCHUNK, 4, 128), dtype=np.int32)
    for w in range(_NW):
        for c in range(_NCHUNK):
            flat = np.zeros((nslots,), dtype=np.int64)
            for s in range(nidx):
                b = w * _B_PER_W + c * _CB + s // K
                flat[s] = b * _ROWS_PER_BATCH + rows_u[s % K]
            gidx[w, c] = flat.reshape(4, 128).astype(np.int32)

    erow = np.zeros((_NVEC, _L), dtype=np.int32)
    ecol = np.zeros((_NVEC, _L), dtype=np.int32)
    for t in range(_OUT_PER_CHUNK):
        b, r = divmod(t, _OUT_PER_B)
        a, cc = divmod(r, _C)
        e = int(_VERT_IDX[a]) * _C + cc
        erow[t // _L, t % _L] = b * K + pos_of[e // _L]
        ecol[t // _L, t % _L] = e % _L
    return gidx, erow, ecol, K


_GIDX_NP, _EROW_NP, _ECOL_NP, _K = _build_static()

_mesh = plsc.VectorSubcoreMesh(core_axis_name="c", subcore_axis_name="s")


@functools.partial(
    pl.kernel,
    out_type=jax.ShapeDtypeStruct((_B * _OUT_PER_B,), jnp.float32),
    mesh=_mesh,
    scratch_types=[
        pltpu.VMEM((_NCHUNK, 4, 128), jnp.int32),
        pltpu.VMEM((_NVEC, _L), jnp.int32),
        pltpu.VMEM((_NVEC, _L), jnp.int32),
        pltpu.VMEM((512, _L), jnp.float32),
        pltpu.VMEM((512, _L), jnp.float32),
        pltpu.VMEM((_OUT_PER_W,), jnp.float32),
        pltpu.SemaphoreType.DMA,
        pltpu.SemaphoreType.DMA,
    ],
    compiler_params=pltpu.CompilerParams(use_tc_tiling_on_sc=False,
                                         needs_layout_passes=False),
)
def _gather_kernel(table, gidx, erow, ecol, out, gidx_v, erow_v, ecol_v,
                   buf0, buf1, stage, sem0, sem1):
    wid = lax.axis_index("s") * 2 + lax.axis_index("c")
    pltpu.sync_copy(gidx.at[wid], gidx_v)
    pltpu.sync_copy(erow, erow_v)
    pltpu.sync_copy(ecol, ecol_v)

    def fire(c, buf, sem):
        for d in range(4):
            pltpu.async_copy(
                table.at[gidx_v.at[c, d]],
                buf.at[pl.ds(d * 128, 128)],
                sem,
            )

    def wait4(c, buf, sem):
        for d in range(4):
            pltpu.make_async_copy(
                table.at[gidx_v.at[c, d]],
                buf.at[pl.ds(d * 128, 128)],
                sem,
            ).wait()

    def extract(c, buf):
        base = c * _OUT_PER_CHUNK
        for k in range(_NVEC):
            v = plsc.load_gather(buf, [erow_v[k], ecol_v[k]])
            stage[pl.ds(base + k * _L, _L)] = v

    fire(0, buf0, sem0)

    def body(i, carry):
        c0 = 2 * i
        c1 = c0 + 1
        wait4(c0, buf0, sem0)
        fire(c1, buf1, sem1)
        extract(c0, buf0)
        wait4(c1, buf1, sem1)

        @pl.when(i < _NCHUNK // 2 - 1)
        def _():
            fire(c0 + 2, buf0, sem0)

        extract(c1, buf1)
        return carry

    lax.fori_loop(0, _NCHUNK // 2, body, 0)

    pltpu.sync_copy(stage, out.at[pl.ds(wid * _OUT_PER_W, _OUT_PER_W)])


def kernel(vertices):
    table = vertices.reshape(_B * _V * _C // _L, _L)
    out = _gather_kernel(table, jnp.asarray(_GIDX_NP), jnp.asarray(_EROW_NP),
                         jnp.asarray(_ECOL_NP))
    return out.reshape(_B, _A, _C)


# speedup vs baseline: 130.8139x; 130.8139x over previous
import functools

import jax
import jax.numpy as jnp
import numpy as np
from jax import lax
from jax.experimental import pallas as pl
from jax.experimental.pallas import tpu as pltpu
from jax.experimental.pallas import tpu_sc as plsc

_VERT_IDX = np.array([
    4646, 4779, 5143, 5109, 5112, 3207, 2391, 5398, 5786, 5925, 5831,
    5895, 2158, 6208, 6428, 6585, 6615, 6620, 2039, 2828, 6783, 7158,
    7407, 7308, 7368, 3820, 3536, 7707, 7856, 8051, 8056, 8063, 5669,
    5891, 5780, 5740, 6468, 6554, 6412, 6297, 7214, 7389, 7122, 7144,
    7975, 8059
], dtype=np.int64)

_B = 4096
_V = 8064
_C = 3
_A = 46
_L = 16
_NVT = _V // 128
_NSLAB = _B * _NVT
_NW = 32
_B_PER_W = _B // _NW
_NCHUNK = _B_PER_W
_OUT_PER_B = _A * _C
_NVEC = -(-_OUT_PER_B // _L)
_OUT_PER_W = _B_PER_W * _OUT_PER_B


def _build_static():
    vtu = np.unique(_VERT_IDX // 128)
    U = len(vtu)
    pos_of = {int(v): p for p, v in enumerate(vtu)}

    gidx = np.zeros((_NW, _NCHUNK, U), dtype=np.int32)
    for w in range(_NW):
        for c in range(_NCHUNK):
            b = w * _B_PER_W + c
            gidx[w, c] = b * _NVT + vtu

    eslab = np.zeros((_NVEC, _L), dtype=np.int32)
    esub = np.zeros((_NVEC, _L), dtype=np.int32)
    elane = np.zeros((_NVEC, _L), dtype=np.int32)
    for t in range(_OUT_PER_B):
        a, cc = divmod(t, _C)
        v = int(_VERT_IDX[a])
        eslab[t // _L, t % _L] = pos_of[v // 128]
        esub[t // _L, t % _L] = cc
        elane[t // _L, t % _L] = v % 128
    return gidx, eslab, esub, elane, U


_GIDX_NP, _ESLAB_NP, _ESUB_NP, _ELANE_NP, _U = _build_static()

_mesh = plsc.VectorSubcoreMesh(core_axis_name="c", subcore_axis_name="s")


@functools.partial(
    pl.kernel,
    out_type=jax.ShapeDtypeStruct((_B * _OUT_PER_B,), jnp.float32),
    mesh=_mesh,
    scratch_types=[
        pltpu.VMEM((_NCHUNK, _U), jnp.int32),
        pltpu.VMEM((_NVEC, _L), jnp.int32),
        pltpu.VMEM((_NVEC, _L), jnp.int32),
        pltpu.VMEM((_NVEC, _L), jnp.int32),
        pltpu.VMEM((_U, 8, 128), jnp.float32),
        pltpu.VMEM((_U, 8, 128), jnp.float32),
        pltpu.VMEM((_OUT_PER_W + _L,), jnp.float32),
        pltpu.SemaphoreType.DMA,
        pltpu.SemaphoreType.DMA,
    ],
    compiler_params=pltpu.CompilerParams(use_tc_tiling_on_sc=True,
                                         needs_layout_passes=False),
)
def _gather_kernel(table, gidx, eslab, esub, elane, out, gidx_v, eslab_v,
                   esub_v, elane_v, buf0, buf1, stage, sem0, sem1):
    wid = lax.axis_index("s") * 2 + lax.axis_index("c")
    pltpu.sync_copy(gidx.at[wid], gidx_v)
    pltpu.sync_copy(eslab, eslab_v)
    pltpu.sync_copy(esub, esub_v)
    pltpu.sync_copy(elane, elane_v)
    lanes = lax.iota(jnp.int32, _L)

    def fire(c, buf, sem):
        pltpu.async_copy(table.at[gidx_v.at[c]],
                         buf.at[:, pl.ds(0, _C), :], sem)

    def wait(c, buf, sem):
        pltpu.make_async_copy(table.at[gidx_v.at[c]],
                              buf.at[:, pl.ds(0, _C), :], sem).wait()

    def extract(c, buf):
        base = c * _OUT_PER_B
        for k in range(_NVEC):
            v = plsc.load_gather(buf, [eslab_v[k], esub_v[k], elane_v[k]])
            plsc.store_scatter(stage, [lanes + (base + k * _L)], v)

    fire(0, buf0, sem0)
    fire(1, buf1, sem1)

    def body(i, carry):
        c0 = 2 * i
        c1 = c0 + 1
        wait(c0, buf0, sem0)
        extract(c0, buf0)

        @pl.when(i < _NCHUNK // 2 - 1)
        def _():
            fire(c0 + 2, buf0, sem0)

        wait(c1, buf1, sem1)
        extract(c1, buf1)

        @pl.when(i < _NCHUNK // 2 - 1)
        def _():
            fire(c1 + 2, buf1, sem1)

        return carry

    lax.fori_loop(0, _NCHUNK // 2, body, 0)

    pltpu.sync_copy(stage.at[pl.ds(0, _OUT_PER_W)],
                    out.at[pl.ds(wid * _OUT_PER_W, _OUT_PER_W)])


def kernel(vertices):
    table = (vertices.reshape(_B, _NVT, 128, _C)
             .transpose(0, 1, 3, 2)
             .reshape(_NSLAB, _C, 128))
    out = _gather_kernel(table, jnp.asarray(_GIDX_NP), jnp.asarray(_ESLAB_NP),
                         jnp.asarray(_ESUB_NP), jnp.asarray(_ELANE_NP))
    return out.reshape(_B, _A, _C)


# speedup vs baseline: 999.8576x; 7.6434x over previous
import functools

import jax
import jax.numpy as jnp
import numpy as np
from jax import lax
from jax.experimental import pallas as pl
from jax.experimental.pallas import tpu as pltpu
from jax.experimental.pallas import tpu_sc as plsc

_VERT_IDX = np.array([
    4646, 4779, 5143, 5109, 5112, 3207, 2391, 5398, 5786, 5925, 5831,
    5895, 2158, 6208, 6428, 6585, 6615, 6620, 2039, 2828, 6783, 7158,
    7407, 7308, 7368, 3820, 3536, 7707, 7856, 8051, 8056, 8063, 5669,
    5891, 5780, 5740, 6468, 6554, 6412, 6297, 7214, 7389, 7122, 7144,
    7975, 8059
], dtype=np.int64)

_B = 4096
_V = 8064
_C = 3
_A = 46
_L = 16
_NVT = _V // 128
_NBT = _B // 8
_NSLAB = _C * _NBT * _NVT
_NW = 32
_B_PER_W = _B // _NW
_BT_PER_W = _B_PER_W // 8
_NCHUNK = _BT_PER_W * _C
_OUT_PER_B = _A * _C
_OUT_PER_BT = 8 * _OUT_PER_B
_NVEC = 8 * _A // _L


def _build_static():
    vtu = np.unique(_VERT_IDX // 128)
    U = len(vtu)
    pos_of = {int(v): p for p, v in enumerate(vtu)}

    gidx = np.zeros((_NW, _NCHUNK, U), dtype=np.int32)
    for w in range(_NW):
        for c in range(_NCHUNK):
            bt = w * _BT_PER_W + c // 3
            cc = c % 3
            gidx[w, c] = (cc * _NBT + bt) * _NVT + vtu

    srow = np.zeros((_NVEC, _L), dtype=np.int32)
    ssub = np.zeros((_NVEC, _L), dtype=np.int32)
    slane = np.zeros((_NVEC, _L), dtype=np.int32)
    danc = np.zeros((_NVEC, _L), dtype=np.int32)
    dbat = np.zeros((_NVEC, _L), dtype=np.int32)
    for t in range(_NVEC * _L):
        bl, a = divmod(t, _A)
        v = int(_VERT_IDX[a])
        srow[t // _L, t % _L] = pos_of[v // 128]
        ssub[t // _L, t % _L] = bl
        slane[t // _L, t % _L] = v % 128
        danc[t // _L, t % _L] = a
        dbat[t // _L, t % _L] = bl
    return gidx, srow, ssub, slane, danc, dbat, U


(_GIDX_NP, _SROW_NP, _SSUB_NP, _SLANE_NP, _DANC_NP, _DBAT_NP,
 _U) = _build_static()

_mesh = plsc.VectorSubcoreMesh(core_axis_name="c", subcore_axis_name="s")


@functools.partial(
    pl.kernel,
    out_type=jax.ShapeDtypeStruct((_C, _A, _B), jnp.float32),
    mesh=_mesh,
    scratch_types=[
        pltpu.VMEM((_NCHUNK, _U), jnp.int32),
        pltpu.VMEM((_NVEC, _L), jnp.int32),
        pltpu.VMEM((_NVEC, _L), jnp.int32),
        pltpu.VMEM((_NVEC, _L), jnp.int32),
        pltpu.VMEM((_NVEC, _L), jnp.int32),
        pltpu.VMEM((_NVEC, _L), jnp.int32),
        pltpu.VMEM((_U, 8, 128), jnp.float32),
        pltpu.VMEM((_U, 8, 128), jnp.float32),
        pltpu.VMEM((_C, 48, 128), jnp.float32),
        pltpu.SemaphoreType.DMA,
        pltpu.SemaphoreType.DMA,
    ],
    compiler_params=pltpu.CompilerParams(use_tc_tiling_on_sc=True,
                                         needs_layout_passes=False),
)
def _gather_kernel(table, gidx, srow, ssub, slane, edanc, edbat, out,
                   gidx_v, srow_v, ssub_v, slane_v, danc_v, dbat_v,
                   buf0, buf1, stage, sem0, sem1):
    wid = lax.axis_index("s") * 2 + lax.axis_index("c")
    pltpu.sync_copy(gidx.at[wid], gidx_v)
    pltpu.sync_copy(srow, srow_v)
    pltpu.sync_copy(ssub, ssub_v)
    pltpu.sync_copy(slane, slane_v)
    pltpu.sync_copy(edanc, danc_v)
    pltpu.sync_copy(edbat, dbat_v)

    def fire(c, buf, sem):
        pltpu.async_copy(table.at[gidx_v.at[c]], buf, sem)

    def wait(c, buf, sem):
        pltpu.make_async_copy(table.at[gidx_v.at[c]], buf, sem).wait()

    def extract(c, buf):
        bt8 = (c // 3) * 8
        cc = c % 3
        for k in range(_NVEC):
            v = plsc.load_gather(buf, [srow_v[k], ssub_v[k], slane_v[k]])
            plsc.store_scatter(
                stage, [danc_v[k] * 0 + cc, danc_v[k], dbat_v[k] + bt8], v)

    fire(0, buf0, sem0)
    fire(1, buf1, sem1)

    def body(i, carry):
        c0 = 2 * i
        c1 = c0 + 1
        wait(c0, buf0, sem0)
        extract(c0, buf0)

        @pl.when(i < _NCHUNK // 2 - 1)
        def _():
            fire(c0 + 2, buf0, sem0)

        wait(c1, buf1, sem1)
        extract(c1, buf1)

        @pl.when(i < _NCHUNK // 2 - 1)
        def _():
            fire(c1 + 2, buf1, sem1)

        return carry

    lax.fori_loop(0, _NCHUNK // 2, body, 0)

    pltpu.sync_copy(stage.at[:, pl.ds(0, _A), :],
                    out.at[:, :, pl.ds(wid * _B_PER_W, _B_PER_W)])


def kernel(vertices):
    table = (vertices.transpose(2, 0, 1)
             .reshape(_C, _NBT, 8, _NVT, 128)
             .transpose(0, 1, 3, 2, 4)
             .reshape(_NSLAB, 8, 128))
    out = _gather_kernel(table, jnp.asarray(_GIDX_NP), jnp.asarray(_SROW_NP),
                         jnp.asarray(_SSUB_NP), jnp.asarray(_SLANE_NP),
                         jnp.asarray(_DANC_NP), jnp.asarray(_DBAT_NP))
    return out.transpose(2, 1, 0)


# speedup vs baseline: 1487.0365x; 1.4872x over previous
import functools

import jax
import jax.numpy as jnp
import numpy as np
from jax import lax
from jax.experimental import pallas as pl
from jax.experimental.pallas import tpu as pltpu
from jax.experimental.pallas import tpu_sc as plsc

_VERT_IDX = np.array([
    4646, 4779, 5143, 5109, 5112, 3207, 2391, 5398, 5786, 5925, 5831,
    5895, 2158, 6208, 6428, 6585, 6615, 6620, 2039, 2828, 6783, 7158,
    7407, 7308, 7368, 3820, 3536, 7707, 7856, 8051, 8056, 8063, 5669,
    5891, 5780, 5740, 6468, 6554, 6412, 6297, 7214, 7389, 7122, 7144,
    7975, 8059
], dtype=np.int64)

_B = 4096
_V = 8064
_C = 3
_A = 46
_L = 16
_NVT = _V // 128
_NBT = _B // 8
_NROW = _B * _V * _C // _L
_NW = 32
_B_PER_W = _B // _NW
_BT_PER_W = _B_PER_W // 8
_NCHUNK = _BT_PER_W * _C
_NVEC = 8 * _A // _L


def _build_static():
    u16 = np.unique(_VERT_IDX // _L)
    P = len(u16)
    pos_of = {int(g): p for p, g in enumerate(u16)}
    nrows = 8 * P
    nd = nrows // 3

    gidx = np.zeros((_NW, _NCHUNK, 3, nd), dtype=np.int32)
    for w in range(_NW):
        for c in range(_NCHUNK):
            bt = w * _BT_PER_W + c // 3
            cc = c % 3
            flat = np.zeros((nrows,), dtype=np.int64)
            for s in range(nrows):
                p, bsub = divmod(s, 8)
                vt, g = divmod(int(u16[p]), 8)
                flat[s] = (((cc * _NBT + bt) * _NVT + vt) * 8 + bsub) * 8 + g
            gidx[w, c] = flat.reshape(3, nd).astype(np.int32)

    srow = np.zeros((_NVEC, _L), dtype=np.int32)
    scol = np.zeros((_NVEC, _L), dtype=np.int32)
    danc = np.zeros((_NVEC, _L), dtype=np.int32)
    dbat = np.zeros((_NVEC, _L), dtype=np.int32)
    for t in range(_NVEC * _L):
        bl, a = divmod(t, _A)
        v = int(_VERT_IDX[a])
        srow[t // _L, t % _L] = pos_of[v // _L] * 8 + bl
        scol[t // _L, t % _L] = v % _L
        danc[t // _L, t % _L] = a
        dbat[t // _L, t % _L] = bl
    return gidx, srow, scol, danc, dbat, nrows, nd


(_GIDX_NP, _SROW_NP, _SCOL_NP, _DANC_NP, _DBAT_NP,
 _NROWS_CHUNK, _ND) = _build_static()

_mesh = plsc.VectorSubcoreMesh(core_axis_name="c", subcore_axis_name="s")


@functools.partial(
    pl.kernel,
    out_type=jax.ShapeDtypeStruct((_C, _A, _B), jnp.float32),
    mesh=_mesh,
    scratch_types=[
        pltpu.VMEM((_NCHUNK, 3, _ND), jnp.int32),
        pltpu.VMEM((_NVEC, _L), jnp.int32),
        pltpu.VMEM((_NVEC, _L), jnp.int32),
        pltpu.VMEM((_NVEC, _L), jnp.int32),
        pltpu.VMEM((_NVEC, _L), jnp.int32),
        pltpu.VMEM((_NROWS_CHUNK, _L), jnp.float32),
        pltpu.VMEM((_NROWS_CHUNK, _L), jnp.float32),
        pltpu.VMEM((_C, _A, 128), jnp.float32),
        pltpu.SemaphoreType.DMA,
        pltpu.SemaphoreType.DMA,
    ],
    compiler_params=pltpu.CompilerParams(use_tc_tiling_on_sc=False,
                                         needs_layout_passes=False),
)
def _gather_kernel(table, gidx, srow, scol, edanc, edbat, out,
                   gidx_v, srow_v, scol_v, danc_v, dbat_v,
                   buf0, buf1, stage, sem0, sem1):
    wid = lax.axis_index("s") * 2 + lax.axis_index("c")
    pltpu.sync_copy(gidx.at[wid], gidx_v)
    pltpu.sync_copy(srow, srow_v)
    pltpu.sync_copy(scol, scol_v)
    pltpu.sync_copy(edanc, danc_v)
    pltpu.sync_copy(edbat, dbat_v)

    def fire(c, buf, sem):
        for d in range(3):
            pltpu.async_copy(table.at[gidx_v.at[c, d]],
                             buf.at[pl.ds(d * _ND, _ND)], sem)

    def wait(c, buf, sem):
        for d in range(3):
            pltpu.make_async_copy(table.at[gidx_v.at[c, d]],
                                  buf.at[pl.ds(d * _ND, _ND)], sem).wait()

    def extract(c, buf):
        bt8 = (c // 3) * 8
        cc = c % 3
        for k in range(_NVEC):
            v = plsc.load_gather(buf, [srow_v[k], scol_v[k]])
            plsc.store_scatter(
                stage, [danc_v[k] * 0 + cc, danc_v[k], dbat_v[k] + bt8], v)

    fire(0, buf0, sem0)
    fire(1, buf1, sem1)

    def body(i, carry):
        c0 = 2 * i
        c1 = c0 + 1
        wait(c0, buf0, sem0)
        extract(c0, buf0)

        @pl.when(i < _NCHUNK // 2 - 1)
        def _():
            fire(c0 + 2, buf0, sem0)

        wait(c1, buf1, sem1)
        extract(c1, buf1)

        @pl.when(i < _NCHUNK // 2 - 1)
        def _():
            fire(c1 + 2, buf1, sem1)

        return carry

    lax.fori_loop(0, _NCHUNK // 2, body, 0)

    pltpu.sync_copy(stage, out.at[:, :, pl.ds(wid * _B_PER_W, _B_PER_W)])


def kernel(vertices):
    table = (vertices.transpose(2, 0, 1)
             .reshape(_C, _NBT, 8, _NVT, 128)
             .transpose(0, 1, 3, 2, 4)
             .reshape(_NROW, _L))
    out = _gather_kernel(table, jnp.asarray(_GIDX_NP), jnp.asarray(_SROW_NP),
                         jnp.asarray(_SCOL_NP), jnp.asarray(_DANC_NP),
                         jnp.asarray(_DBAT_NP))
    return out.transpose(2, 1, 0)
